# Initial kernel scaffold; baseline (speedup 1.0000x reference)
import functools

import jax
import jax.numpy as jnp
from jax import lax
from jax.experimental import pallas as pl
from jax.experimental.pallas import tpu as pltpu
from jax.experimental.pallas import tpu_sc as plsc

N_NODES = 10000
D_FEAT = 256
D_HALF = 128
N_EDGES = 160000

NC = 2
NS = 16
CHUNK = 128
CHUNKS_PER_TILE = 79
E_PAD = NS * CHUNKS_PER_TILE * CHUNK
AGG_ROWS = 10240
ROWS_PER_TILE = AGG_ROWS // NS
OUT_ROWS_PER_TILE = N_NODES // NS
ZROWS = 64


def _sc_body(xa_hbm, xb_hbm, row_hbm, col_hbm,
             agg_a_hbm, agg_b_hbm, deg_hbm,
             agg_sh, deg_sh, dst_idx, col_idx, gbuf, ones_v, zbuf):
    c = lax.axis_index("c")
    s = lax.axis_index("s")

    def _zfill(j, _):
        i = j // 8
        k = j % 8
        zbuf[i, pl.ds(k * 16, 16)] = jnp.zeros((16,), jnp.float32)
        return 0
    lax.fori_loop(0, ZROWS * 8, _zfill, 0)

    def _onefill(j, _):
        ones_v[j, :] = jnp.ones((16,), jnp.float32)
        return 0
    lax.fori_loop(0, CHUNK, _onefill, 0)

    base = s * ROWS_PER_TILE
    for r in range(ROWS_PER_TILE // ZROWS):
        pltpu.sync_copy(zbuf, agg_sh.at[pl.ds(base + r * ZROWS, ZROWS)])
        pltpu.sync_copy(zbuf.at[:, pl.ds(0, 16)],
                        deg_sh.at[pl.ds(base + r * ZROWS, ZROWS)])

    pltpu.sync_copy(row_hbm.at[s], dst_idx)
    pltpu.sync_copy(col_hbm.at[s], col_idx)

    plsc.subcore_barrier()

    def _edge_loop(x_hbm, with_deg):
        def body(j, _):
            pltpu.sync_copy(x_hbm.at[col_idx.at[j]], gbuf)
            pltpu.sync_copy(gbuf, agg_sh.at[dst_idx.at[j]], add=True)
            if with_deg:
                pltpu.sync_copy(ones_v, deg_sh.at[dst_idx.at[j]], add=True)
            return 0
        lax.fori_loop(0, CHUNKS_PER_TILE, body, 0)

    @pl.when(c == 0)
    def _():
        _edge_loop(xa_hbm, True)

    @pl.when(c == 1)
    def _():
        _edge_loop(xb_hbm, False)

    plsc.subcore_barrier()

    obase = s * OUT_ROWS_PER_TILE
    osl = pl.ds(obase, OUT_ROWS_PER_TILE)

    @pl.when(c == 0)
    def _():
        pltpu.sync_copy(agg_sh.at[osl], agg_a_hbm.at[osl])
        pltpu.sync_copy(deg_sh.at[osl], deg_hbm.at[osl])

    @pl.when(c == 1)
    def _():
        pltpu.sync_copy(agg_sh.at[osl], agg_b_hbm.at[osl])


def _sc_aggregate(xa, xb, row_r, col_r):
    mesh = plsc.VectorSubcoreMesh(core_axis_name="c", subcore_axis_name="s",
                                  num_cores=NC, num_subcores=NS)
    f32 = jnp.float32
    return pl.kernel(
        _sc_body,
        out_type=(
            jax.ShapeDtypeStruct((N_NODES, D_HALF), f32),
            jax.ShapeDtypeStruct((N_NODES, D_HALF), f32),
            jax.ShapeDtypeStruct((N_NODES, 16), f32),
        ),
        mesh=mesh,
        scratch_types=(
            pltpu.VMEM_SHARED((AGG_ROWS, D_HALF), f32),
            pltpu.VMEM_SHARED((AGG_ROWS, 16), f32),
            pltpu.VMEM((CHUNKS_PER_TILE, CHUNK), jnp.int32),
            pltpu.VMEM((CHUNKS_PER_TILE, CHUNK), jnp.int32),
            pltpu.VMEM((CHUNK, D_HALF), f32),
            pltpu.VMEM((CHUNK, 16), f32),
            pltpu.VMEM((ZROWS, D_HALF), f32),
        ),
        name="graphconv_sc_aggregate",
    )(xa, xb, row_r, col_r)


def _mm_body(agg_a_ref, agg_b_ref, deg_ref, wat_ref, wbt_ref, b_ref, out_ref):
    deg = jnp.maximum(deg_ref[:, 0:1], 1.0)
    acc = jnp.dot(agg_a_ref[...], wat_ref[...],
                  preferred_element_type=jnp.float32)
    acc += jnp.dot(agg_b_ref[...], wbt_ref[...],
                   preferred_element_type=jnp.float32)
    out_ref[...] = acc / deg + b_ref[...]


def _tc_linear(agg_a, agg_b, deg, wat, wbt, b2d):
    blk = 2000
    grid = (N_NODES // blk,)
    return pl.pallas_call(
        _mm_body,
        grid=grid,
        in_specs=[
            pl.BlockSpec((blk, D_HALF), lambda i: (i, 0)),
            pl.BlockSpec((blk, D_HALF), lambda i: (i, 0)),
            pl.BlockSpec((blk, 16), lambda i: (i, 0)),
            pl.BlockSpec((D_HALF, D_FEAT), lambda i: (0, 0)),
            pl.BlockSpec((D_HALF, D_FEAT), lambda i: (0, 0)),
            pl.BlockSpec((1, D_FEAT), lambda i: (0, 0)),
        ],
        out_specs=pl.BlockSpec((blk, D_FEAT), lambda i: (i, 0)),
        out_shape=jax.ShapeDtypeStruct((N_NODES, D_FEAT), jnp.float32),
        name="graphconv_tc_linear",
    )(agg_a, agg_b, deg, wat, wbt, b2d)


def kernel(x, edge_index, W, b):
    ei = edge_index.astype(jnp.int32)
    row = ei[0]
    col = ei[1]
    pad = E_PAD - N_EDGES
    row_p = jnp.concatenate([row, jnp.full((pad,), N_NODES, jnp.int32)])
    col_p = jnp.concatenate([col, jnp.zeros((pad,), jnp.int32)])
    row_r = row_p.reshape(NS, CHUNKS_PER_TILE, CHUNK)
    col_r = col_p.reshape(NS, CHUNKS_PER_TILE, CHUNK)

    xa = x[:, :D_HALF]
    xb = x[:, D_HALF:]
    agg_a, agg_b, deg = _sc_aggregate(xa, xb, row_r, col_r)

    wat = W[:, :D_HALF].T
    wbt = W[:, D_HALF:].T
    b2d = b.reshape(1, D_FEAT)
    return _tc_linear(agg_a, agg_b, deg, wat, wbt, b2d)


# speedup vs baseline: 3.8808x; 3.8808x over previous
import jax
import jax.numpy as jnp
from jax import lax
from jax.experimental import pallas as pl
from jax.experimental.pallas import tpu as pltpu
from jax.experimental.pallas import tpu_sc as plsc

N_NODES = 10000
D_FEAT = 256
D_HALF = 128
N_EDGES = 160000

NC = 2
NS = 16
CHUNK = 128
IBLK = 8
NBLK = 10
CHUNKS_PER_TILE = IBLK * NBLK
E_PAD = NS * CHUNKS_PER_TILE * CHUNK
AGG_ROWS = 10240
RPT = AGG_ROWS // NS
ZROWS = 16


def _sc_body(xa_hbm, xb_hbm, row_hbm, col_hbm,
             agg_a_hbm, agg_b_hbm, deg_hbm,
             agg_sh, deg_stage, dst_blk, col_blk, gbuf, zbuf,
             hist, dbuf, rbuf):
    c = lax.axis_index("c")
    s = lax.axis_index("s")

    def _zfill(j, _):
        zbuf[j // 8, pl.ds((j % 8) * 16, 16)] = jnp.zeros((16,), jnp.float32)
        return 0
    lax.fori_loop(0, ZROWS * 8, _zfill, 0)

    base = s * RPT

    def _zcopy(r, _):
        pltpu.sync_copy(zbuf, agg_sh.at[pl.ds(base + r * ZROWS, ZROWS)])
        return 0
    lax.fori_loop(0, RPT // ZROWS, _zcopy, 0)

    @pl.when(c == 0)
    def _():
        def _hzero(j, _):
            hist[pl.ds(j * 16, 16)] = jnp.zeros((16,), jnp.float32)
            return 0
        lax.fori_loop(0, AGG_ROWS // 16, _hzero, 0)

    plsc.subcore_barrier()

    ones16 = jnp.ones((16,), jnp.float32)

    def _edge_loop(x_hbm, with_deg):
        def blk_body(jj, _):
            pltpu.sync_copy(row_hbm.at[s, pl.ds(jj * IBLK, IBLK)], dst_blk)
            pltpu.sync_copy(col_hbm.at[s, pl.ds(jj * IBLK, IBLK)], col_blk)
            for k in range(IBLK):
                pltpu.sync_copy(x_hbm.at[col_blk.at[k]], gbuf)
                pltpu.sync_copy(gbuf, agg_sh.at[dst_blk.at[k]], add=True)
                if with_deg:
                    for i in range(CHUNK // 16):
                        d = dst_blk[k, pl.ds(i * 16, 16)]
                        plsc.addupdate_scatter(hist, [d], ones16)
            return 0
        lax.fori_loop(0, NBLK, blk_body, 0)

    @pl.when(c == 0)
    def _():
        _edge_loop(xa_hbm, True)

    @pl.when(c == 1)
    def _():
        _edge_loop(xb_hbm, False)

    @pl.when(c == 0)
    def _():
        pltpu.sync_copy(hist, deg_stage.at[s])

    plsc.subcore_barrier()

    osl = pl.ds(s * RPT, RPT)

    @pl.when(c == 0)
    def _():
        pltpu.sync_copy(agg_sh.at[osl], agg_a_hbm.at[osl])

    @pl.when(c == 1)
    def _():
        pltpu.sync_copy(agg_sh.at[osl], agg_b_hbm.at[osl])

    @pl.when(c == 0)
    def _():
        dsl = pl.ds(s * RPT, RPT)
        pltpu.sync_copy(deg_stage.at[0, dsl], rbuf)
        for k in range(1, NS):
            pltpu.sync_copy(deg_stage.at[k, dsl], dbuf)

            def _acc(v, _):
                rbuf[pl.ds(v * 16, 16)] = (rbuf[pl.ds(v * 16, 16)]
                                           + dbuf[pl.ds(v * 16, 16)])
                return 0
            lax.fori_loop(0, RPT // 16, _acc, 0)
        pltpu.sync_copy(rbuf, deg_hbm.at[dsl])


def _sc_aggregate(xa, xb, row_r, col_r):
    mesh = plsc.VectorSubcoreMesh(core_axis_name="c", subcore_axis_name="s",
                                  num_cores=NC, num_subcores=NS)
    f32 = jnp.float32
    return pl.kernel(
        _sc_body,
        out_type=(
            jax.ShapeDtypeStruct((AGG_ROWS, D_HALF), f32),
            jax.ShapeDtypeStruct((AGG_ROWS, D_HALF), f32),
            jax.ShapeDtypeStruct((AGG_ROWS,), f32),
        ),
        mesh=mesh,
        scratch_types=(
            pltpu.VMEM_SHARED((AGG_ROWS, D_HALF), f32),
            pltpu.VMEM_SHARED((NS, AGG_ROWS), f32),
            pltpu.VMEM((IBLK, CHUNK), jnp.int32),
            pltpu.VMEM((IBLK, CHUNK), jnp.int32),
            pltpu.VMEM((CHUNK, D_HALF), f32),
            pltpu.VMEM((ZROWS, D_HALF), f32),
            pltpu.VMEM((AGG_ROWS,), f32),
            pltpu.VMEM((RPT,), f32),
            pltpu.VMEM((RPT,), f32),
        ),
        name="graphconv_sc_aggregate",
        compiler_params=pltpu.CompilerParams(needs_layout_passes=False),
    )(xa, xb, row_r, col_r)


def _mm_body(agg_a_ref, agg_b_ref, deg_ref, wat_ref, wbt_ref, b_ref, out_ref):
    deg = jnp.maximum(deg_ref[...], 1.0)
    acc = jnp.dot(agg_a_ref[...], wat_ref[...],
                  preferred_element_type=jnp.float32)
    acc += jnp.dot(agg_b_ref[...], wbt_ref[...],
                   preferred_element_type=jnp.float32)
    out_ref[...] = acc / deg + b_ref[...]


def _tc_linear(agg_a, agg_b, deg2d, wat, wbt, b2d):
    blk = 2048
    grid = (AGG_ROWS // blk,)
    return pl.pallas_call(
        _mm_body,
        grid=grid,
        in_specs=[
            pl.BlockSpec((blk, D_HALF), lambda i: (i, 0)),
            pl.BlockSpec((blk, D_HALF), lambda i: (i, 0)),
            pl.BlockSpec((blk, 1), lambda i: (i, 0)),
            pl.BlockSpec((D_HALF, D_FEAT), lambda i: (0, 0)),
            pl.BlockSpec((D_HALF, D_FEAT), lambda i: (0, 0)),
            pl.BlockSpec((1, D_FEAT), lambda i: (0, 0)),
        ],
        out_specs=pl.BlockSpec((blk, D_FEAT), lambda i: (i, 0)),
        out_shape=jax.ShapeDtypeStruct((N_NODES, D_FEAT), jnp.float32),
        name="graphconv_tc_linear",
    )(agg_a, agg_b, deg2d, wat, wbt, b2d)


def kernel(x, edge_index, W, b):
    ei = edge_index.astype(jnp.int32)
    row = ei[0]
    col = ei[1]
    pad = E_PAD - N_EDGES
    row_p = jnp.concatenate([row, jnp.full((pad,), N_NODES, jnp.int32)])
    col_p = jnp.concatenate([col, jnp.zeros((pad,), jnp.int32)])
    row_r = row_p.reshape(NS, CHUNKS_PER_TILE, CHUNK)
    col_r = col_p.reshape(NS, CHUNKS_PER_TILE, CHUNK)

    xa = x[:, :D_HALF]
    xb = x[:, D_HALF:]
    agg_a, agg_b, deg = _sc_aggregate(xa, xb, row_r, col_r)

    deg2d = deg.reshape(AGG_ROWS, 1)
    wat = W[:, :D_HALF].T
    wbt = W[:, D_HALF:].T
    b2d = b.reshape(1, D_FEAT)
    return _tc_linear(agg_a, agg_b, deg2d, wat, wbt, b2d)


# speedup vs baseline: 4.3548x; 1.1221x over previous
import jax
import jax.numpy as jnp
from jax import lax
from jax.experimental import pallas as pl
from jax.experimental.pallas import tpu as pltpu
from jax.experimental.pallas import tpu_sc as plsc

N_NODES = 10000
D_FEAT = 256
D_HALF = 128
N_EDGES = 160000

NC = 2
NS = 16
CHUNK = 128
IBLK = 8
NBLK = 10
CPT = IBLK * NBLK
E_PAD = NS * CPT * CHUNK
AGG_ROWS = 10240
HIST_ROWS = 10016
RPT = AGG_ROWS // NS


def _sc_body(x2_hbm, row_hbm, col_hbm, zeros_hbm,
             agg2_hbm, deg_part_hbm,
             agg_sh, dst_blk, col_blk, gbuf, hist,
             sem_g, sem_id, sem_ic):
    c = lax.axis_index("c")
    s = lax.axis_index("s")

    pltpu.sync_copy(zeros_hbm, agg_sh.at[pl.ds(s * RPT, RPT)])

    @pl.when(c == 0)
    def _():
        def _hzero(j, _):
            hist[pl.ds(j * 16, 16)] = jnp.zeros((16,), jnp.float32)
            return 0
        lax.fori_loop(0, HIST_ROWS // 16, _hzero, 0)

    pltpu.async_copy(row_hbm.at[s, pl.ds(0, IBLK)], dst_blk.at[0], sem_id).wait()
    pltpu.async_copy(col_hbm.at[s, pl.ds(0, IBLK)], col_blk.at[0], sem_ic).wait()

    plsc.subcore_barrier()

    ones16 = jnp.ones((16,), jnp.float32)
    pltpu.async_copy(x2_hbm.at[c].at[col_blk.at[0, 0]], gbuf.at[0], sem_g)

    def chunk_body(j, _):
        blk = j >> 3
        k = j & 7
        p = blk & 1
        q = j & 1
        j1 = j + 1
        p1 = (j1 >> 3) & 1
        k1 = j1 & 7

        @pl.when((k == 0) & (blk + 1 < NBLK))
        def _():
            off = (blk + 1) * IBLK
            pltpu.async_copy(row_hbm.at[s, pl.ds(off, IBLK)],
                             dst_blk.at[1 - p], sem_id)
            pltpu.async_copy(col_hbm.at[s, pl.ds(off, IBLK)],
                             col_blk.at[1 - p], sem_ic)

        @pl.when((k == 7) & (j1 < CPT))
        def _():
            pltpu.make_async_copy(row_hbm.at[s, pl.ds(0, IBLK)],
                                  dst_blk.at[1 - p], sem_id).wait()
            pltpu.make_async_copy(col_hbm.at[s, pl.ds(0, IBLK)],
                                  col_blk.at[1 - p], sem_ic).wait()

        pltpu.make_async_copy(x2_hbm.at[c, pl.ds(0, CHUNK)],
                              gbuf.at[q], sem_g).wait()

        @pl.when(j1 < CPT)
        def _():
            pltpu.async_copy(x2_hbm.at[c].at[col_blk.at[p1, k1]],
                             gbuf.at[1 - q], sem_g)

        pltpu.sync_copy(gbuf.at[q], agg_sh.at[dst_blk.at[p, k]], add=True)

        @pl.when(c == 0)
        def _():
            def _h(i, _):
                d = dst_blk[p, k, pl.ds(i * 16, 16)]
                plsc.addupdate_scatter(hist, [d], ones16)
                return 0
            lax.fori_loop(0, CHUNK // 16, _h, 0)

        return 0

    lax.fori_loop(0, CPT, chunk_body, 0)

    @pl.when(c == 0)
    def _():
        pltpu.sync_copy(hist, deg_part_hbm.at[s])

    plsc.subcore_barrier()

    osl = pl.ds(s * RPT, RPT)
    pltpu.sync_copy(agg_sh.at[osl], agg2_hbm.at[c, osl])


def _sc_aggregate(x2, row_r, col_r, zeros_rows):
    mesh = plsc.VectorSubcoreMesh(core_axis_name="c", subcore_axis_name="s",
                                  num_cores=NC, num_subcores=NS)
    f32 = jnp.float32
    return pl.kernel(
        _sc_body,
        out_type=(
            jax.ShapeDtypeStruct((NC, AGG_ROWS, D_HALF), f32),
            jax.ShapeDtypeStruct((NS, HIST_ROWS), f32),
        ),
        mesh=mesh,
        scratch_types=(
            pltpu.VMEM_SHARED((AGG_ROWS, D_HALF), f32),
            pltpu.VMEM((2, IBLK, CHUNK), jnp.int32),
            pltpu.VMEM((2, IBLK, CHUNK), jnp.int32),
            pltpu.VMEM((2, CHUNK, D_HALF), f32),
            pltpu.VMEM((HIST_ROWS,), f32),
            pltpu.SemaphoreType.DMA,
            pltpu.SemaphoreType.DMA,
            pltpu.SemaphoreType.DMA,
        ),
        name="graphconv_sc_aggregate",
        compiler_params=pltpu.CompilerParams(needs_layout_passes=False),
    )(x2, row_r, col_r, zeros_rows)


def _mm_body(agg_a_ref, agg_b_ref, deg_ref, wat_ref, wbt_ref, b_ref, out_ref):
    deg = jnp.maximum(jnp.sum(deg_ref[...], axis=1, keepdims=True), 1.0)
    acc = jnp.dot(agg_a_ref[0], wat_ref[...],
                  preferred_element_type=jnp.float32)
    acc += jnp.dot(agg_b_ref[0], wbt_ref[...],
                   preferred_element_type=jnp.float32)
    out_ref[...] = acc / deg + b_ref[...]


def _tc_linear(agg2, deg_t, wat, wbt, b2d):
    blk = 2048
    grid = (AGG_ROWS // blk,)
    return pl.pallas_call(
        _mm_body,
        grid=grid,
        in_specs=[
            pl.BlockSpec((1, blk, D_HALF), lambda i: (0, i, 0)),
            pl.BlockSpec((1, blk, D_HALF), lambda i: (1, i, 0)),
            pl.BlockSpec((blk, NS), lambda i: (i, 0)),
            pl.BlockSpec((D_HALF, D_FEAT), lambda i: (0, 0)),
            pl.BlockSpec((D_HALF, D_FEAT), lambda i: (0, 0)),
            pl.BlockSpec((1, D_FEAT), lambda i: (0, 0)),
        ],
        out_specs=pl.BlockSpec((blk, D_FEAT), lambda i: (i, 0)),
        out_shape=jax.ShapeDtypeStruct((N_NODES, D_FEAT), jnp.float32),
        name="graphconv_tc_linear",
    )(agg2, agg2, deg_t, wat, wbt, b2d)


def kernel(x, edge_index, W, b):
    ei = edge_index.astype(jnp.int32)
    row = ei[0]
    col = ei[1]
    pad = E_PAD - N_EDGES
    row_p = jnp.concatenate([row, jnp.full((pad,), N_NODES, jnp.int32)])
    col_p = jnp.concatenate([col, jnp.zeros((pad,), jnp.int32)])
    row_r = row_p.reshape(NS, CPT, CHUNK)
    col_r = col_p.reshape(NS, CPT, CHUNK)

    x2 = jnp.stack([x[:, :D_HALF], x[:, D_HALF:]])
    zeros_rows = jnp.zeros((RPT, D_HALF), jnp.float32)
    agg2, deg_part = _sc_aggregate(x2, row_r, col_r, zeros_rows)

    deg_t = deg_part.T
    wat = W[:, :D_HALF].T
    wbt = W[:, D_HALF:].T
    b2d = b.reshape(1, D_FEAT)
    return _tc_linear(agg2, deg_t, wat, wbt, b2d)
